# baseline (speedup 1.0000x reference)
import jax
import jax.numpy as jnp
from jax import lax
from jax.experimental import pallas as pl

_F32 = jnp.float32
_BB = 4


def _bdot(x, y):
    return jnp.dot(x, y, preferred_element_type=_F32)


def _st_stage(xs, a, ones_row, ones_col, prms):
    uv = [lax.dot_general(x, p[2], (((1,), (1,)), ((), ())),
                          preferred_element_type=_F32)
          for x, p in zip(xs, prms)]
    u = [p[3][0, 0] * t[:, 0:1] for t, p in zip(uv, prms)]
    v = [t[:, 1:2] for t in uv]
    mh = [jnp.max(uu) * jnp.maximum(vv, 0.0)
          + jnp.min(uu) * jnp.minimum(vv, 0.0)
          for uu, vv in zip(u, v)]
    u2 = [jnp.concatenate([uu, jnp.full_like(uu, -1.0)], axis=1)
          for uu in u]
    v2 = [jnp.concatenate([vv, mm], axis=1)
          for vv, mm in zip(v, mh)]
    s0 = [lax.dot_general(a2, b2, (((1,), (1,)), ((), ())),
                          preferred_element_type=_F32) + p[1]
          for a2, b2, p in zip(u2, v2, prms)]
    e1 = [jnp.exp(s) for s in s0]
    r1 = [1.0 / _colsum(ones_row, e) for e in e1]
    s2 = [_bdot(p[0], e) * r for p, e, r in zip(prms, e1, r1)]
    e2 = [jnp.exp(s) for s in s2]
    r2 = [1.0 / _colsum(ones_row, e) for e in e2]
    ah = [a * e * r for e, r in zip(e2, r2)]
    deg = [_bdot(h, ones_col) for h in ah]
    dinv = [jnp.where(d > 0.0, lax.rsqrt(jnp.where(d > 0.0, d, 1.0)), 0.0)
            for d in deg]
    tx1 = [-di * _bdot(h, di * x) for di, h, x in zip(dinv, ah, xs)]
    tx2 = [-2.0 * di * _bdot(h, di * t) - x
           for di, h, t, x in zip(dinv, ah, tx1, xs)]
    out = [_bdot(x, p[4][0]) + _bdot(t1, p[4][1]) + _bdot(t2, p[4][2])
           + p[5]
           for x, t1, t2, p in zip(xs, tx1, tx2, prms)]
    xn = [jnp.maximum(o, 0.0) for o in out]
    h = xn[0].shape[1]
    r = lax.broadcasted_iota(jnp.int32, (h, h), 0)
    c = lax.broadcasted_iota(jnp.int32, (h, h), 1)
    cmat = [(jnp.where(c == r + 1, p[3][0, 1], 0.0)
             + jnp.where(c == r, p[3][0, 2], 0.0)
             + jnp.where(c == r - 1, p[3][0, 3], 0.0)) for p in prms]
    return [jnp.maximum(_bdot(x, cm) + p[3][0, 4], 0.0)
            for x, cm, p in zip(xn, cmat, prms)]


def _colsum(ones_row, s):
    return _bdot(ones_row, s)


def _fwd_body(xh_ref, xd_ref, xw_ref, a_ref, *rest):
    out_ref = rest[-1]
    prefs = rest[:-1]
    a = a_ref[...]
    n = a.shape[0]
    ones_row = jnp.full((1, n), 1.0, dtype=_F32)
    ones_col = jnp.full((n, 1), 1.0, dtype=_F32)
    per_branch = 2 * 6 + 2
    layer_prms = [[], []]
    lins = []
    for bi in range(3):
        base = bi * per_branch
        for li in range(2):
            layer_prms[li].append(tuple(
                r[...] for r in prefs[base + li * 6: base + (li + 1) * 6]))
        lins.append((prefs[base + 12][...], prefs[base + 13][...]))
    xs = [r[i] for i in range(_BB) for r in (xh_ref, xd_ref, xw_ref)]
    for li in range(2):
        prm6 = layer_prms[li] * _BB
        xs = _st_stage(xs, a, ones_row, ones_col, prm6)
    lin6 = lins * _BB
    ys = [jnp.maximum(_bdot(x, lw) + lb, 0.0)
          for x, (lw, lb) in zip(xs, lin6)]
    fus = prefs[-1][...]
    for i in range(_BB):
        yh, yd, yw = ys[3 * i: 3 * i + 3]
        out_ref[i] = fus[0:1] * yh + fus[1:2] * yw + fus[2:3] * yd


def kernel(Xh, Xd, Xw, A, params):
    B, Nn = Xh.shape[0], Xh.shape[1]
    Tp = params['H']['lin']['W'].shape[1]

    xs = [X.reshape(B, Nn, X.shape[-1]).astype(_F32) for X in (Xh, Xd, Xw)]

    ops = []
    specs = []

    def add_const(arr):
        if arr.dtype != jnp.bfloat16:
            arr = arr.astype(_F32)
        ops.append(arr)
        nd = arr.ndim
        specs.append(pl.BlockSpec(arr.shape, lambda b, _nd=nd: (0,) * _nd))

    x_specs = [pl.BlockSpec((_BB, Nn, x.shape[-1]), lambda b: (b, 0, 0))
               for x in xs]

    add_const(A)
    for key in ('H', 'D', 'W'):
        bp = params[key]
        for lp in bp['layers']:
            sat = lp['satt']
            add_const(sat['Vs'])
            add_const(sat['bs'] - jnp.max(sat['bs']))
            add_const(jnp.stack([sat['W1'], sat['W2'][0]]))
            add_const(jnp.concatenate(
                [sat['W3'], lp['conv_w'],
                 jnp.reshape(lp['conv_b'], (1,))])[None])
            add_const(lp['cheb_W'])
            add_const(lp['cheb_b'][None])
        add_const(bp['lin']['W'])
        add_const(bp['lin']['b'][None])
    f = params['fusion']
    add_const(jnp.stack([f['Wh'], f['Wd'], f['Ww']]))

    out = pl.pallas_call(
        _fwd_body,
        grid=(B // _BB,),
        in_specs=x_specs + specs,
        out_specs=pl.BlockSpec((_BB, Nn, Tp), lambda b: (b, 0, 0)),
        out_shape=jax.ShapeDtypeStruct((B, Nn, Tp), _F32),
    )(*xs, *ops)
    return out.reshape(B, Nn, 1, Tp)


# speedup vs baseline: 1.2334x; 1.2334x over previous
import jax
import jax.numpy as jnp
from jax import lax
from jax.experimental import pallas as pl

_F32 = jnp.float32
_BB = 4


def _bdot(x, y):
    return jnp.dot(x, y, preferred_element_type=_F32)


def _st_stage(xs, a, ones_row, ones_col, prms):
    uv = [lax.dot_general(x, p[2], (((1,), (1,)), ((), ())),
                          preferred_element_type=_F32)
          for x, p in zip(xs, prms)]
    s0 = [lax.dot_general(p[3][0, 0] * t[:, 0:1], t[:, 1:2],
                          (((1,), (1,)), ((), ())),
                          preferred_element_type=_F32) + p[1]
          for t, p in zip(uv, prms)]
    m1 = [jnp.max(s, axis=0, keepdims=True) for s in s0]
    e1 = [jnp.exp(s - m) for s, m in zip(s0, m1)]
    r1 = [1.0 / _colsum(ones_row, e) for e in e1]
    s2 = [_bdot(p[0], e) * r for p, e, r in zip(prms, e1, r1)]
    e2 = [jnp.exp(s) for s in s2]
    r2 = [1.0 / _colsum(ones_row, e) for e in e2]
    ah = [a * e * r for e, r in zip(e2, r2)]
    deg = [_bdot(h, ones_col) for h in ah]
    dinv = [jnp.where(d > 0.0, lax.rsqrt(jnp.where(d > 0.0, d, 1.0)), 0.0)
            for d in deg]
    tx1 = [-di * _bdot(h, di * x) for di, h, x in zip(dinv, ah, xs)]
    tx2 = [-2.0 * di * _bdot(h, di * t) - x
           for di, h, t, x in zip(dinv, ah, tx1, xs)]
    out = [_bdot(x, p[4][0]) + _bdot(t1, p[4][1]) + _bdot(t2, p[4][2])
           + p[5]
           for x, t1, t2, p in zip(xs, tx1, tx2, prms)]
    xn = [jnp.maximum(o, 0.0) for o in out]
    h = xn[0].shape[1]
    r = lax.broadcasted_iota(jnp.int32, (h, h), 0)
    c = lax.broadcasted_iota(jnp.int32, (h, h), 1)
    cmat = [(jnp.where(c == r + 1, p[3][0, 1], 0.0)
             + jnp.where(c == r, p[3][0, 2], 0.0)
             + jnp.where(c == r - 1, p[3][0, 3], 0.0)) for p in prms]
    return [jnp.maximum(_bdot(x, cm) + p[3][0, 4], 0.0)
            for x, cm, p in zip(xn, cmat, prms)]


def _colsum(ones_row, s):
    return _bdot(ones_row, s)


def _fwd_body(xh_ref, xd_ref, xw_ref, a_ref, *rest):
    out_ref = rest[-1]
    prefs = rest[:-1]
    a = a_ref[...]
    n = a.shape[0]
    ones_row = jnp.full((1, n), 1.0, dtype=_F32)
    ones_col = jnp.full((n, 1), 1.0, dtype=_F32)
    per_branch = 2 * 6 + 2
    layer_prms = [[], []]
    lins = []
    for bi in range(3):
        base = bi * per_branch
        for li in range(2):
            layer_prms[li].append(tuple(
                r[...] for r in prefs[base + li * 6: base + (li + 1) * 6]))
        lins.append((prefs[base + 12][...], prefs[base + 13][...]))
    xs = [r[i] for i in range(_BB) for r in (xh_ref, xd_ref, xw_ref)]
    for li in range(2):
        prm6 = layer_prms[li] * _BB
        xs = _st_stage(xs, a, ones_row, ones_col, prm6)
    lin6 = lins * _BB
    ys = [jnp.maximum(_bdot(x, lw) + lb, 0.0)
          for x, (lw, lb) in zip(xs, lin6)]
    fus = prefs[-1][...]
    for i in range(_BB):
        yh, yd, yw = ys[3 * i: 3 * i + 3]
        out_ref[i] = fus[0:1] * yh + fus[1:2] * yw + fus[2:3] * yd


def kernel(Xh, Xd, Xw, A, params):
    B, Nn = Xh.shape[0], Xh.shape[1]
    Tp = params['H']['lin']['W'].shape[1]

    xs = [X.reshape(B, Nn, X.shape[-1]).astype(_F32) for X in (Xh, Xd, Xw)]

    ops = []
    specs = []

    def add_const(arr):
        if arr.dtype != jnp.bfloat16:
            arr = arr.astype(_F32)
        ops.append(arr)
        nd = arr.ndim
        specs.append(pl.BlockSpec(arr.shape, lambda b, _nd=nd: (0,) * _nd))

    x_specs = [pl.BlockSpec((_BB, Nn, x.shape[-1]), lambda b: (b, 0, 0))
               for x in xs]

    add_const(A)
    for key in ('H', 'D', 'W'):
        bp = params[key]
        for lp in bp['layers']:
            sat = lp['satt']
            add_const(sat['Vs'])
            add_const(sat['bs'])
            add_const(jnp.stack([sat['W1'], sat['W2'][0]]))
            add_const(jnp.concatenate(
                [sat['W3'], lp['conv_w'],
                 jnp.reshape(lp['conv_b'], (1,))])[None])
            add_const(lp['cheb_W'])
            add_const(lp['cheb_b'][None])
        add_const(bp['lin']['W'])
        add_const(bp['lin']['b'][None])
    f = params['fusion']
    add_const(jnp.stack([f['Wh'], f['Wd'], f['Ww']]))

    out = pl.pallas_call(
        _fwd_body,
        grid=(B // _BB,),
        in_specs=x_specs + specs,
        out_specs=pl.BlockSpec((_BB, Nn, Tp), lambda b: (b, 0, 0)),
        out_shape=jax.ShapeDtypeStruct((B, Nn, Tp), _F32),
    )(*xs, *ops)
    return out.reshape(B, Nn, 1, Tp)
